# Initial kernel scaffold; baseline (speedup 1.0000x reference)
import jax
import jax.numpy as jnp
from jax.experimental import pallas as pl


def kernel(embeddings, labels):
    raise NotImplementedError("write your pallas kernel here")


# speedup vs baseline: 1.4977x; 1.4977x over previous
import functools

import jax
import jax.numpy as jnp
from jax.experimental import pallas as pl
from jax.experimental.pallas import tpu as pltpu

MARGIN_C = 0.3
EPS_C = 1e-6
NEG_INF_C = -1e30
POS_INF_C = 1e30
BIG_IDX = 2 ** 30


def _mine_body(lab_row_ref, lab_col_ref, emb_ref, out_ref, acc_ref, *, bm, nsteps):
    i = pl.program_id(0)
    emb = emb_ref[...]
    n, d = emb.shape
    sq = jnp.sum(emb * emb, axis=1)
    s = jnp.sum(emb, axis=1)

    x = emb_ref[pl.ds(i * bm, bm), :]
    sq_blk = jnp.sum(x * x, axis=1)
    s_blk = jnp.sum(x, axis=1)

    g = jax.lax.dot_general(x, emb, (((1,), (1,)), ((), ())),
                            preferred_element_type=jnp.float32)
    d2 = (sq_blk[:, None] + sq[None, :]) - 2.0 * g
    dist = jnp.sqrt(jnp.maximum(d2, 0.0))

    lab_blk = lab_col_ref[pl.ds(i * bm, bm), :]
    lab_all = lab_row_ref[...]
    same = lab_blk == lab_all
    cidx = jax.lax.broadcasted_iota(jnp.int32, (bm, n), 1)
    ridx = i * bm + jax.lax.broadcasted_iota(jnp.int32, (bm, n), 0)
    pos = same & (cidx != ridx)
    neg = jnp.logical_not(same)

    ap = jnp.where(pos, dist, NEG_INF_C)
    ap_max = jnp.max(ap, axis=1)
    semi = neg & (dist > ap_max[:, None]) & (dist < ap_max[:, None] + MARGIN_C)
    first_semi = jnp.min(jnp.where(semi, cidx, BIG_IDX), axis=1)
    has_semi = first_semi < BIG_IDX
    an = jnp.where(neg, dist, POS_INF_C)
    an_min = jnp.min(an, axis=1)
    hn_idx = jnp.min(jnp.where(neg & (dist == an_min[:, None]), cidx, BIG_IDX),
                     axis=1)
    neg_idx = jnp.where(has_semi, first_semi, hn_idx)

    corr2 = d2 + (2.0 * EPS_C) * (s_blk[:, None] - s[None, :]) + (d * EPS_C * EPS_C)
    corr = jnp.sqrt(jnp.maximum(corr2, 0.0))
    d_ap = jnp.max(jnp.where(pos & (dist == ap_max[:, None]), corr, NEG_INF_C),
                   axis=1)
    d_an = jnp.max(jnp.where(cidx == neg_idx[:, None], corr, NEG_INF_C), axis=1)

    valid = (ap_max >= 0.0) & (an_min < POS_INF_C)
    per = jnp.maximum(d_ap - d_an + MARGIN_C, 0.0)
    tot = jnp.sum(jnp.where(valid, per, 0.0))
    cnt = jnp.sum(valid.astype(jnp.float32))

    @pl.when(i == 0)
    def _init():
        acc_ref[0] = tot
        acc_ref[1] = cnt

    @pl.when(i > 0)
    def _accum():
        acc_ref[0] += tot
        acc_ref[1] += cnt

    @pl.when(i == nsteps - 1)
    def _finish():
        t = acc_ref[0]
        c = acc_ref[1]
        res = jnp.where(c > 0.0, t / jnp.maximum(c, 1.0), jnp.float32(0.0))
        out_ref[...] = jnp.full((1, 1), res, jnp.float32)


@jax.jit
def kernel(embeddings, labels):
    n, d = embeddings.shape
    bm = 256
    nsteps = n // bm
    lab_row = labels.reshape(1, n).astype(jnp.int32)
    lab_col = labels.reshape(n, 1).astype(jnp.int32)
    out = pl.pallas_call(
        functools.partial(_mine_body, bm=bm, nsteps=nsteps),
        grid=(nsteps,),
        in_specs=[
            pl.BlockSpec((1, n), lambda i: (0, 0)),
            pl.BlockSpec((n, 1), lambda i: (0, 0)),
            pl.BlockSpec((n, d), lambda i: (0, 0)),
        ],
        out_specs=pl.BlockSpec((1, 1), lambda i: (0, 0)),
        out_shape=jax.ShapeDtypeStruct((1, 1), jnp.float32),
        scratch_shapes=[pltpu.SMEM((2,), jnp.float32)],
    )(lab_row, lab_col, embeddings)
    return out[0, 0]


# speedup vs baseline: 2.4611x; 1.6432x over previous
import functools

import jax
import jax.numpy as jnp
from jax.experimental import pallas as pl
from jax.experimental.pallas import tpu as pltpu

MARGIN_C = 0.3
EPS_C = 1e-6
NEG_INF_C = -1e30
POS_INF_C = 1e30
BIG_IDX = 2 ** 30


def _mine_body(lab_row_ref, lab_col_ref, emb_ref, out_ref, acc_ref, *, bm, nsteps):
    i = pl.program_id(0)
    emb = emb_ref[...]
    n, d = emb.shape
    sq = jnp.sum(emb * emb, axis=1)

    x = emb_ref[pl.ds(i * bm, bm), :]
    sq_blk = jnp.sum(x * x, axis=1)

    g2 = jax.lax.dot_general(-2.0 * x, emb, (((1,), (1,)), ((), ())),
                             preferred_element_type=jnp.float32)
    d2 = (sq_blk[:, None] + sq[None, :]) + g2
    dist = jnp.sqrt(jnp.maximum(d2, 0.0))

    lab_blk = lab_col_ref[pl.ds(i * bm, bm), :]
    lab_all = lab_row_ref[...]
    same = lab_blk == lab_all
    cidx = jax.lax.broadcasted_iota(jnp.int32, (bm, n), 1)
    ridx = i * bm + jax.lax.broadcasted_iota(jnp.int32, (bm, n), 0)
    pos = same & (cidx != ridx)

    ap = jnp.where(pos, dist, NEG_INF_C)
    ap_max = jnp.max(ap, axis=1)
    semi = (jnp.logical_not(same)
            & (dist > ap_max[:, None]) & (dist < ap_max[:, None] + MARGIN_C))
    first_semi = jnp.min(jnp.where(semi, cidx, BIG_IDX), axis=1)
    has_semi = first_semi < BIG_IDX
    d_semi = jnp.max(jnp.where(cidx == first_semi[:, None], dist, NEG_INF_C),
                     axis=1)
    an_min = jnp.min(jnp.where(same, POS_INF_C, dist), axis=1)

    d_an = jnp.where(has_semi, d_semi, an_min)
    valid = (ap_max >= 0.0) & (an_min < POS_INF_C)
    per = jnp.maximum(ap_max - d_an + MARGIN_C, 0.0)
    tot = jnp.sum(jnp.where(valid, per, 0.0))
    cnt = jnp.sum(valid.astype(jnp.float32))

    @pl.when(i == 0)
    def _init():
        acc_ref[0] = tot
        acc_ref[1] = cnt

    @pl.when(i > 0)
    def _accum():
        acc_ref[0] += tot
        acc_ref[1] += cnt

    @pl.when(i == nsteps - 1)
    def _finish():
        t = acc_ref[0]
        c = acc_ref[1]
        res = jnp.where(c > 0.0, t / jnp.maximum(c, 1.0), jnp.float32(0.0))
        out_ref[...] = jnp.full((1, 1), res, jnp.float32)


@jax.jit
def kernel(embeddings, labels):
    n, d = embeddings.shape
    bm = 256
    nsteps = n // bm
    lab_row = labels.reshape(1, n).astype(jnp.int32)
    lab_col = labels.reshape(n, 1).astype(jnp.int32)
    out = pl.pallas_call(
        functools.partial(_mine_body, bm=bm, nsteps=nsteps),
        grid=(nsteps,),
        in_specs=[
            pl.BlockSpec((1, n), lambda i: (0, 0)),
            pl.BlockSpec((n, 1), lambda i: (0, 0)),
            pl.BlockSpec((n, d), lambda i: (0, 0)),
        ],
        out_specs=pl.BlockSpec((1, 1), lambda i: (0, 0)),
        out_shape=jax.ShapeDtypeStruct((1, 1), jnp.float32),
        scratch_shapes=[pltpu.SMEM((2,), jnp.float32)],
    )(lab_row, lab_col, embeddings)
    return out[0, 0]


# speedup vs baseline: 3.5073x; 1.4251x over previous
import functools

import jax
import jax.numpy as jnp
from jax.experimental import pallas as pl
from jax.experimental.pallas import tpu as pltpu

MARGIN_C = 0.3
NEG_INF_C = -1e30
POS_INF_C = 1e30
BIG_IDX = 2 ** 30


def _mine_body(lab_row_ref, lab_col_ref, cidx_ref, emb_ref, out_ref,
               sq_scr, acc_ref, *, bm, nsteps):
    i = pl.program_id(0)
    n, d = emb_ref.shape

    @pl.when(i == 0)
    def _stage():
        emb = emb_ref[...]
        sq_scr[...] = jnp.sum(emb * emb, axis=1)[None, :]

    x = emb_ref[pl.ds(i * bm, bm), :]
    sq_blk = jnp.sum(x * x, axis=1)

    g2 = jax.lax.dot_general(-2.0 * x, emb_ref[...], (((1,), (1,)), ((), ())),
                             preferred_element_type=jnp.float32)
    d2 = jnp.maximum((sq_blk[:, None] + sq_scr[...]) + g2, 0.0)

    lab_blk = lab_col_ref[pl.ds(i * bm, bm), :]
    same = lab_blk == lab_row_ref[...]
    cnt_same = jnp.sum(same.astype(jnp.float32), axis=1)

    ap2 = jnp.max(jnp.where(same, d2, NEG_INF_C), axis=1)
    ap_d = jnp.sqrt(jnp.maximum(ap2, 0.0))
    hi = ap_d + MARGIN_C
    hi2 = hi * hi
    semi = (d2 > ap2[:, None]) & (d2 < hi2[:, None])
    first_semi = jnp.min(jnp.where(semi, cidx_ref[...], BIG_IDX), axis=1)
    has_semi = first_semi < BIG_IDX
    d2_semi = jnp.max(jnp.where(cidx_ref[...] == first_semi[:, None],
                                d2, NEG_INF_C), axis=1)
    an2 = jnp.min(jnp.where(same, POS_INF_C, d2), axis=1)

    d_an = jnp.where(has_semi, jnp.sqrt(jnp.maximum(d2_semi, 0.0)),
                     jnp.sqrt(an2))
    valid = (cnt_same >= 2.0) & (cnt_same <= n - 1.0)
    per = jnp.maximum(ap_d - d_an + MARGIN_C, 0.0)
    tot = jnp.sum(jnp.where(valid, per, 0.0))
    cnt = jnp.sum(valid.astype(jnp.float32))

    @pl.when(i == 0)
    def _init():
        acc_ref[0] = tot
        acc_ref[1] = cnt

    @pl.when(i > 0)
    def _accum():
        acc_ref[0] += tot
        acc_ref[1] += cnt

    @pl.when(i == nsteps - 1)
    def _finish():
        t = acc_ref[0]
        c = acc_ref[1]
        res = jnp.where(c > 0.0, t / jnp.maximum(c, 1.0), jnp.float32(0.0))
        out_ref[...] = jnp.full((1, 1), res, jnp.float32)


@jax.jit
def kernel(embeddings, labels):
    n, d = embeddings.shape
    bm = 256
    nsteps = n // bm
    lab_row = labels.reshape(1, n).astype(jnp.int32)
    lab_col = labels.reshape(n, 1).astype(jnp.int32)
    cidx = jax.lax.broadcasted_iota(jnp.int32, (1, n), 1)
    out = pl.pallas_call(
        functools.partial(_mine_body, bm=bm, nsteps=nsteps),
        grid=(nsteps,),
        in_specs=[
            pl.BlockSpec((1, n), lambda i: (0, 0)),
            pl.BlockSpec((n, 1), lambda i: (0, 0)),
            pl.BlockSpec((1, n), lambda i: (0, 0)),
            pl.BlockSpec((n, d), lambda i: (0, 0)),
        ],
        out_specs=pl.BlockSpec((1, 1), lambda i: (0, 0)),
        out_shape=jax.ShapeDtypeStruct((1, 1), jnp.float32),
        scratch_shapes=[
            pltpu.VMEM((1, n), jnp.float32),
            pltpu.SMEM((2,), jnp.float32),
        ],
    )(lab_row, lab_col, cidx, embeddings)
    return out[0, 0]


# speedup vs baseline: 3.5824x; 1.0214x over previous
import functools

import jax
import jax.numpy as jnp
from jax.experimental import pallas as pl
from jax.experimental.pallas import tpu as pltpu

MARGIN_C = 0.3
NEG_INF_C = -1e30
POS_INF_C = 1e30
BIG_IDX = 2 ** 30


def _mine_body(lab_row_ref, lab_col_ref, cidx_ref, emb_ref, out_ref,
               sq_scr, acc_ref, *, bm, nsteps):
    i = pl.program_id(0)
    n, d = emb_ref.shape
    emb = emb_ref[...]

    @pl.when(i == 0)
    def _stage():
        sq_scr[...] = jnp.sum(emb * emb, axis=1)[None, :]

    x = emb_ref[pl.ds(i * bm, bm), :]
    sq_blk = jnp.sum(x * x, axis=1)

    g2 = jax.lax.dot_general(-2.0 * x, emb, (((1,), (1,)), ((), ())),
                             preferred_element_type=jnp.float32)
    e2 = sq_scr[...] + g2

    lab_blk = lab_col_ref[pl.ds(i * bm, bm), :]
    same = lab_blk == lab_row_ref[...]
    cnt_same = jnp.sum(same.astype(jnp.float32), axis=1)

    ap2e = jnp.max(jnp.where(same, e2, NEG_INF_C), axis=1)
    ap_d = jnp.sqrt(jnp.maximum(ap2e + sq_blk, 0.0))
    hi = ap_d + MARGIN_C
    hi2e = hi * hi - sq_blk
    semi = (e2 > ap2e[:, None]) & (e2 < hi2e[:, None])
    first_semi = jnp.min(jnp.where(semi, cidx_ref[...], BIG_IDX), axis=1)
    has_semi = first_semi < BIG_IDX
    e2_semi = jnp.max(jnp.where(cidx_ref[...] == first_semi[:, None],
                                e2, NEG_INF_C), axis=1)
    an2e = jnp.min(jnp.where(same, POS_INF_C, e2), axis=1)

    d_an = jnp.where(has_semi,
                     jnp.sqrt(jnp.maximum(e2_semi + sq_blk, 0.0)),
                     jnp.sqrt(jnp.maximum(an2e + sq_blk, 0.0)))
    valid = (cnt_same >= 2.0) & (cnt_same <= n - 1.0)
    per = jnp.maximum(ap_d - d_an + MARGIN_C, 0.0)
    tot = jnp.sum(jnp.where(valid, per, 0.0))
    cnt = jnp.sum(valid.astype(jnp.float32))

    @pl.when(i == 0)
    def _init():
        acc_ref[0] = tot
        acc_ref[1] = cnt

    @pl.when(i > 0)
    def _accum():
        acc_ref[0] += tot
        acc_ref[1] += cnt

    @pl.when(i == nsteps - 1)
    def _finish():
        t = acc_ref[0]
        c = acc_ref[1]
        res = jnp.where(c > 0.0, t / jnp.maximum(c, 1.0), jnp.float32(0.0))
        out_ref[...] = jnp.full((1, 1), res, jnp.float32)


@jax.jit
def kernel(embeddings, labels):
    n, d = embeddings.shape
    bm = 256
    nsteps = n // bm
    lab_row = labels.reshape(1, n).astype(jnp.int32)
    lab_col = labels.reshape(n, 1).astype(jnp.int32)
    cidx = jax.lax.broadcasted_iota(jnp.int32, (1, n), 1)
    out = pl.pallas_call(
        functools.partial(_mine_body, bm=bm, nsteps=nsteps),
        grid=(nsteps,),
        in_specs=[
            pl.BlockSpec((1, n), lambda i: (0, 0)),
            pl.BlockSpec((n, 1), lambda i: (0, 0)),
            pl.BlockSpec((1, n), lambda i: (0, 0)),
            pl.BlockSpec((n, d), lambda i: (0, 0)),
        ],
        out_specs=pl.BlockSpec((1, 1), lambda i: (0, 0)),
        out_shape=jax.ShapeDtypeStruct((1, 1), jnp.float32),
        scratch_shapes=[
            pltpu.VMEM((1, n), jnp.float32),
            pltpu.SMEM((2,), jnp.float32),
        ],
    )(lab_row, lab_col, cidx, embeddings)
    return out[0, 0]


# speedup vs baseline: 3.8202x; 1.0664x over previous
import functools

import jax
import jax.numpy as jnp
from jax.experimental import pallas as pl
from jax.experimental.pallas import tpu as pltpu

MARGIN_C = 0.3
NEG_INF_C = -1e30
POS_INF_C = 1e30
BIG_IDX = 2 ** 30
NCLS = 128


def _prep_body(emb_ref, lab_row_ref, sq_ref, cnt_ref):
    emb = emb_ref[...]
    sq_ref[...] = jnp.sum(emb * emb, axis=1)[None, :]
    cls = jax.lax.broadcasted_iota(jnp.int32, (NCLS, 1), 0)
    onehot = (lab_row_ref[...] == cls).astype(jnp.float32)
    cnt_ref[...] = jnp.sum(onehot, axis=1, keepdims=True)


def _mine_body(lab_row_ref, lab_col_ref, cidx_ref, sq_ref, cnt_ref, emb_ref,
               out_ref, acc_ref, *, bm, nsteps):
    i = pl.program_id(0)
    n, d = emb_ref.shape
    emb = emb_ref[...]

    x = emb_ref[pl.ds(i * bm, bm), :]
    sq_blk = jnp.sum(x * x, axis=1)

    g2 = jax.lax.dot_general(-2.0 * x, emb, (((1,), (1,)), ((), ())),
                             preferred_element_type=jnp.float32)
    e2 = sq_ref[...] + g2

    lab_blk = lab_col_ref[pl.ds(i * bm, bm), :]
    same = lab_blk == lab_row_ref[...]

    cls_row = jax.lax.broadcasted_iota(jnp.int32, (1, NCLS), 1)
    oh_blk = (lab_blk == cls_row).astype(jnp.float32)
    cnt_same = jax.lax.dot_general(oh_blk, cnt_ref[...], (((1,), (0,)), ((), ())),
                                   preferred_element_type=jnp.float32)[:, 0]

    ap2e = jnp.max(jnp.where(same, e2, NEG_INF_C), axis=1)
    ap_d = jnp.sqrt(jnp.maximum(ap2e + sq_blk, 0.0))
    hi = ap_d + MARGIN_C
    hi2e = hi * hi - sq_blk
    semi = (e2 > ap2e[:, None]) & (e2 < hi2e[:, None])
    first_semi = jnp.min(jnp.where(semi, cidx_ref[...], BIG_IDX), axis=1)
    has_semi = first_semi < BIG_IDX
    e2_semi = jnp.max(jnp.where(cidx_ref[...] == first_semi[:, None],
                                e2, NEG_INF_C), axis=1)
    an2e = jnp.min(jnp.where(same, POS_INF_C, e2), axis=1)

    d_an = jnp.where(has_semi,
                     jnp.sqrt(jnp.maximum(e2_semi + sq_blk, 0.0)),
                     jnp.sqrt(jnp.maximum(an2e + sq_blk, 0.0)))
    valid = (cnt_same >= 2.0) & (cnt_same <= n - 1.0)
    per = jnp.maximum(ap_d - d_an + MARGIN_C, 0.0)
    tot = jnp.sum(jnp.where(valid, per, 0.0))
    cnt = jnp.sum(valid.astype(jnp.float32))

    @pl.when(i == 0)
    def _init():
        acc_ref[0] = tot
        acc_ref[1] = cnt

    @pl.when(i > 0)
    def _accum():
        acc_ref[0] += tot
        acc_ref[1] += cnt

    @pl.when(i == nsteps - 1)
    def _finish():
        t = acc_ref[0]
        c = acc_ref[1]
        res = jnp.where(c > 0.0, t / jnp.maximum(c, 1.0), jnp.float32(0.0))
        out_ref[...] = jnp.full((1, 1), res, jnp.float32)


@jax.jit
def kernel(embeddings, labels):
    n, d = embeddings.shape
    bm = 256
    nsteps = n // bm
    lab_row = labels.reshape(1, n).astype(jnp.int32)
    lab_col = labels.reshape(n, 1).astype(jnp.int32)
    cidx = jax.lax.broadcasted_iota(jnp.int32, (1, n), 1)

    sq_row, cls_cnt = pl.pallas_call(
        _prep_body,
        in_specs=[
            pl.BlockSpec((n, d), lambda: (0, 0)),
            pl.BlockSpec((1, n), lambda: (0, 0)),
        ],
        out_specs=[
            pl.BlockSpec((1, n), lambda: (0, 0)),
            pl.BlockSpec((NCLS, 1), lambda: (0, 0)),
        ],
        out_shape=[
            jax.ShapeDtypeStruct((1, n), jnp.float32),
            jax.ShapeDtypeStruct((NCLS, 1), jnp.float32),
        ],
    )(embeddings, lab_row)

    out = pl.pallas_call(
        functools.partial(_mine_body, bm=bm, nsteps=nsteps),
        grid=(nsteps,),
        in_specs=[
            pl.BlockSpec((1, n), lambda i: (0, 0)),
            pl.BlockSpec((n, 1), lambda i: (0, 0)),
            pl.BlockSpec((1, n), lambda i: (0, 0)),
            pl.BlockSpec((1, n), lambda i: (0, 0)),
            pl.BlockSpec((NCLS, 1), lambda i: (0, 0)),
            pl.BlockSpec((n, d), lambda i: (0, 0)),
        ],
        out_specs=pl.BlockSpec((1, 1), lambda i: (0, 0)),
        out_shape=jax.ShapeDtypeStruct((1, 1), jnp.float32),
        scratch_shapes=[
            pltpu.SMEM((2,), jnp.float32),
        ],
    )(lab_row, lab_col, cidx, sq_row, cls_cnt, embeddings)
    return out[0, 0]


# speedup vs baseline: 3.9021x; 1.0214x over previous
import functools

import jax
import jax.numpy as jnp
from jax.experimental import pallas as pl
from jax.experimental.pallas import tpu as pltpu

MARGIN_C = 0.3
NEG_INF_C = -1e30
POS_INF_C = 1e30
BIG_IDX = 2 ** 30
NCLS = 128


def _prep_body(emb_ref, lab_row_ref, sq_ref, cnt_ref):
    emb = emb_ref[...]
    sq_ref[...] = jnp.sum(emb * emb, axis=1)[None, :]
    cls = jax.lax.broadcasted_iota(jnp.int32, (NCLS, 1), 0)
    onehot = (lab_row_ref[...] == cls).astype(jnp.float32)
    cnt_ref[...] = jnp.sum(onehot, axis=1, keepdims=True)


def _mine_body(lab_row_ref, lab_col_ref, cidx_ref, sq_ref, cnt_ref, emb_ref,
               out_ref, acc_ref, *, bm, nsteps):
    i = pl.program_id(0)
    n, d = emb_ref.shape
    emb = emb_ref[...]

    x = emb_ref[pl.ds(i * bm, bm), :]
    sq_blk = jnp.sum(x * x, axis=1)

    g2 = jax.lax.dot_general(-2.0 * x, emb, (((1,), (1,)), ((), ())),
                             preferred_element_type=jnp.float32)
    e2 = sq_ref[...] + g2

    lab_blk = lab_col_ref[pl.ds(i * bm, bm), :]
    same = lab_blk == lab_row_ref[...]

    cls_row = jax.lax.broadcasted_iota(jnp.int32, (1, NCLS), 1)
    oh_blk = (lab_blk == cls_row).astype(jnp.float32)
    cnt_same = jax.lax.dot_general(oh_blk, cnt_ref[...], (((1,), (0,)), ((), ())),
                                   preferred_element_type=jnp.float32)[:, 0]

    ap2e = jnp.max(jnp.where(same, e2, NEG_INF_C), axis=1)
    ap_d = jnp.sqrt(jnp.maximum(ap2e + sq_blk, 0.0))
    hi = ap_d + MARGIN_C
    hi2e = hi * hi - sq_blk
    semi = (e2 > ap2e[:, None]) & (e2 < hi2e[:, None])
    first_semi = jnp.min(jnp.where(semi, cidx_ref[...], BIG_IDX), axis=1)
    has_semi = first_semi < BIG_IDX
    e2_semi = jnp.max(jnp.where(cidx_ref[...] == first_semi[:, None],
                                e2, NEG_INF_C), axis=1)
    an2e = jnp.min(jnp.where(same, POS_INF_C, e2), axis=1)

    d_an = jnp.where(has_semi,
                     jnp.sqrt(jnp.maximum(e2_semi + sq_blk, 0.0)),
                     jnp.sqrt(jnp.maximum(an2e + sq_blk, 0.0)))
    valid = (cnt_same >= 2.0) & (cnt_same <= n - 1.0)
    per = jnp.maximum(ap_d - d_an + MARGIN_C, 0.0)
    tot = jnp.sum(jnp.where(valid, per, 0.0))
    cnt = jnp.sum(valid.astype(jnp.float32))

    @pl.when(i == 0)
    def _init():
        acc_ref[0] = tot
        acc_ref[1] = cnt

    @pl.when(i > 0)
    def _accum():
        acc_ref[0] += tot
        acc_ref[1] += cnt

    @pl.when(i == nsteps - 1)
    def _finish():
        t = acc_ref[0]
        c = acc_ref[1]
        res = jnp.where(c > 0.0, t / jnp.maximum(c, 1.0), jnp.float32(0.0))
        out_ref[...] = jnp.full((1, 1), res, jnp.float32)


@jax.jit
def kernel(embeddings, labels):
    n, d = embeddings.shape
    bm = 512
    nsteps = n // bm
    lab_row = labels.reshape(1, n).astype(jnp.int32)
    lab_col = labels.reshape(n, 1).astype(jnp.int32)
    cidx = jax.lax.broadcasted_iota(jnp.int32, (1, n), 1)

    sq_row, cls_cnt = pl.pallas_call(
        _prep_body,
        in_specs=[
            pl.BlockSpec((n, d), lambda: (0, 0)),
            pl.BlockSpec((1, n), lambda: (0, 0)),
        ],
        out_specs=[
            pl.BlockSpec((1, n), lambda: (0, 0)),
            pl.BlockSpec((NCLS, 1), lambda: (0, 0)),
        ],
        out_shape=[
            jax.ShapeDtypeStruct((1, n), jnp.float32),
            jax.ShapeDtypeStruct((NCLS, 1), jnp.float32),
        ],
    )(embeddings, lab_row)

    out = pl.pallas_call(
        functools.partial(_mine_body, bm=bm, nsteps=nsteps),
        grid=(nsteps,),
        in_specs=[
            pl.BlockSpec((1, n), lambda i: (0, 0)),
            pl.BlockSpec((n, 1), lambda i: (0, 0)),
            pl.BlockSpec((1, n), lambda i: (0, 0)),
            pl.BlockSpec((1, n), lambda i: (0, 0)),
            pl.BlockSpec((NCLS, 1), lambda i: (0, 0)),
            pl.BlockSpec((n, d), lambda i: (0, 0)),
        ],
        out_specs=pl.BlockSpec((1, 1), lambda i: (0, 0)),
        out_shape=jax.ShapeDtypeStruct((1, 1), jnp.float32),
        scratch_shapes=[
            pltpu.SMEM((2,), jnp.float32),
        ],
    )(lab_row, lab_col, cidx, sq_row, cls_cnt, embeddings)
    return out[0, 0]


# speedup vs baseline: 3.9982x; 1.0246x over previous
import functools

import jax
import jax.numpy as jnp
from jax.experimental import pallas as pl
from jax.experimental.pallas import tpu as pltpu

MARGIN_C = 0.3
NEG_INF_C = -1e30
POS_INF_C = 1e30
BIG_IDX = 2 ** 30
NCLS = 128


def _prep_body(emb_ref, lab_row_ref, sq_ref, cnt_ref):
    emb = emb_ref[...]
    sq_ref[...] = jnp.sum(emb * emb, axis=1)[None, :]
    cls = jax.lax.broadcasted_iota(jnp.int32, (NCLS, 1), 0)
    onehot = (lab_row_ref[...] == cls).astype(jnp.float32)
    cnt_ref[...] = jnp.sum(onehot, axis=1, keepdims=True)


def _mine_body(lab_row_ref, lab_col_ref, cidx_ref, sq_ref, cnt_ref, emb_ref,
               out_ref, acc_ref, *, bm, nsteps):
    i = pl.program_id(0)
    n, d = emb_ref.shape
    emb = emb_ref[...]

    x = emb_ref[pl.ds(i * bm, bm), :]
    sq_blk = jnp.sum(x * x, axis=1)

    g2 = jax.lax.dot_general(-2.0 * x, emb, (((1,), (1,)), ((), ())),
                             preferred_element_type=jnp.float32)
    e2 = sq_ref[...] + g2

    lab_blk = lab_col_ref[pl.ds(i * bm, bm), :]
    same = lab_blk == lab_row_ref[...]

    cls_row = jax.lax.broadcasted_iota(jnp.int32, (1, NCLS), 1)
    oh_blk = (lab_blk == cls_row).astype(jnp.float32)
    cnt_same = jax.lax.dot_general(oh_blk, cnt_ref[...], (((1,), (0,)), ((), ())),
                                   preferred_element_type=jnp.float32)[:, 0]

    ap2e = jnp.max(jnp.where(same, e2, NEG_INF_C), axis=1)
    ap_d = jnp.sqrt(jnp.maximum(ap2e + sq_blk, 0.0))
    hi = ap_d + MARGIN_C
    hi2e = hi * hi - sq_blk
    semi = (e2 > ap2e[:, None]) & (e2 < hi2e[:, None])
    first_semi = jnp.min(jnp.where(semi, cidx_ref[...], BIG_IDX), axis=1)
    has_semi = first_semi < BIG_IDX
    e2_semi = jnp.max(jnp.where(cidx_ref[...] == first_semi[:, None],
                                e2, NEG_INF_C), axis=1)
    an2e = jnp.min(jnp.where(same, POS_INF_C, e2), axis=1)

    d_an = jnp.where(has_semi,
                     jnp.sqrt(jnp.maximum(e2_semi + sq_blk, 0.0)),
                     jnp.sqrt(jnp.maximum(an2e + sq_blk, 0.0)))
    valid = (cnt_same >= 2.0) & (cnt_same <= n - 1.0)
    per = jnp.maximum(ap_d - d_an + MARGIN_C, 0.0)
    tot = jnp.sum(jnp.where(valid, per, 0.0))
    cnt = jnp.sum(valid.astype(jnp.float32))

    @pl.when(i == 0)
    def _init():
        acc_ref[0] = tot
        acc_ref[1] = cnt

    @pl.when(i > 0)
    def _accum():
        acc_ref[0] += tot
        acc_ref[1] += cnt

    @pl.when(i == nsteps - 1)
    def _finish():
        t = acc_ref[0]
        c = acc_ref[1]
        res = jnp.where(c > 0.0, t / jnp.maximum(c, 1.0), jnp.float32(0.0))
        out_ref[...] = jnp.full((1, 1), res, jnp.float32)


@jax.jit
def kernel(embeddings, labels):
    n, d = embeddings.shape
    bm = 1024
    nsteps = n // bm
    lab_row = labels.reshape(1, n).astype(jnp.int32)
    lab_col = labels.reshape(n, 1).astype(jnp.int32)
    cidx = jax.lax.broadcasted_iota(jnp.int32, (1, n), 1)

    sq_row, cls_cnt = pl.pallas_call(
        _prep_body,
        in_specs=[
            pl.BlockSpec((n, d), lambda: (0, 0)),
            pl.BlockSpec((1, n), lambda: (0, 0)),
        ],
        out_specs=[
            pl.BlockSpec((1, n), lambda: (0, 0)),
            pl.BlockSpec((NCLS, 1), lambda: (0, 0)),
        ],
        out_shape=[
            jax.ShapeDtypeStruct((1, n), jnp.float32),
            jax.ShapeDtypeStruct((NCLS, 1), jnp.float32),
        ],
    )(embeddings, lab_row)

    out = pl.pallas_call(
        functools.partial(_mine_body, bm=bm, nsteps=nsteps),
        grid=(nsteps,),
        in_specs=[
            pl.BlockSpec((1, n), lambda i: (0, 0)),
            pl.BlockSpec((n, 1), lambda i: (0, 0)),
            pl.BlockSpec((1, n), lambda i: (0, 0)),
            pl.BlockSpec((1, n), lambda i: (0, 0)),
            pl.BlockSpec((NCLS, 1), lambda i: (0, 0)),
            pl.BlockSpec((n, d), lambda i: (0, 0)),
        ],
        out_specs=pl.BlockSpec((1, 1), lambda i: (0, 0)),
        out_shape=jax.ShapeDtypeStruct((1, 1), jnp.float32),
        scratch_shapes=[
            pltpu.SMEM((2,), jnp.float32),
        ],
    )(lab_row, lab_col, cidx, sq_row, cls_cnt, embeddings)
    return out[0, 0]
